# baseline (speedup 1.0000x reference)
import jax
import jax.numpy as jnp
import numpy as np
from jax import lax
from jax.experimental import pallas as pl
from jax.experimental.pallas import tpu as pltpu
from jax.experimental.pallas import tpu_sc as plsc

_IOU_T = 0.5
_SCORE_T = 0.5
_MAX_OUT = 200
_N = 5000
_NPAD = 5120
_BC = 16
_KPAD = 256
_P = _BC * _MAX_OUT
_PBUF = _P + 256


def _nms_kernel(x1_ref, y1_ref, x2_ref, y2_ref, sc_ref, ts_ref,
                keep_ref, cnt_ref):
    x1 = x1_ref[...]
    y1 = y1_ref[...]
    x2 = x2_ref[...]
    y2 = y2_ref[...]
    sc = sc_ref[...]
    areas = (x2 - x1) * (y2 - y1)

    iota_n = jax.lax.broadcasted_iota(jnp.int32, (_BC, _NPAD), 1)
    col_iota = jax.lax.broadcasted_iota(jnp.int32, (_BC, _KPAD), 1)

    tmat = ts_ref[...]
    mask_hi = jnp.int32(-65536)
    t_hi = jax.lax.bitcast_convert_type(
        jax.lax.bitcast_convert_type(tmat, jnp.int32) & mask_hi, jnp.float32)
    rres = tmat - t_hi
    t_mid = jax.lax.bitcast_convert_type(
        jax.lax.bitcast_convert_type(rres, jnp.int32) & mask_hi, jnp.float32)
    t_lo = rres - t_mid
    rowmask = jax.lax.broadcasted_iota(jnp.int32, (_BC, 1), 0) < 8

    scm0 = jnp.where(sc > _SCORE_T, sc, -1.0)
    keep0 = jnp.zeros((_BC, _KPAD), jnp.int32)
    cnt0 = jnp.zeros((_BC, 1), jnp.int32)

    def body(k, state):
        scm, keep, cnt = state
        m = jnp.max(scm, axis=1, keepdims=True)
        found = m > _SCORE_T
        idx = jnp.min(jnp.where(scm == m, iota_n, _NPAD), axis=1, keepdims=True)
        onehot = iota_n == idx
        onehot_f = onehot.astype(jnp.float32)
        dn = (((1,), (0,)), ((), ()))
        g1 = jax.lax.dot_general(onehot_f, t_hi, dn,
                                 preferred_element_type=jnp.float32)
        g2 = jax.lax.dot_general(onehot_f, t_mid, dn,
                                 preferred_element_type=jnp.float32)
        g3 = jax.lax.dot_general(onehot_f, t_lo, dn,
                                 preferred_element_type=jnp.float32)
        coords = (g1 + g2) + g3
        bx1 = jnp.where(rowmask, coords[:, 0:1], coords[:, 4:5])
        by1 = jnp.where(rowmask, coords[:, 1:2], coords[:, 5:6])
        bx2 = jnp.where(rowmask, coords[:, 2:3], coords[:, 6:7])
        by2 = jnp.where(rowmask, coords[:, 3:4], coords[:, 7:8])
        barea = (bx2 - bx1) * (by2 - by1)
        w = jnp.maximum(jnp.minimum(x2, bx2) - jnp.maximum(x1, bx1), 0.0)
        h = jnp.maximum(jnp.minimum(y2, by2) - jnp.maximum(y1, by1), 0.0)
        inter = w * h
        union = areas + barea - inter
        sup = inter / union > _IOU_T
        scm = jnp.where((sup | onehot) & found, -1.0, scm)
        keep = keep + jnp.where((col_iota == k) & found, idx, 0)
        cnt = cnt + found.astype(jnp.int32)
        return scm, keep, cnt

    _, keep, cnt = jax.lax.fori_loop(0, _MAX_OUT, body, (scm0, keep0, cnt0),
                                     unroll=4)
    keep_ref[...] = keep
    cnt_ref[...] = cnt


def _pack_kernel(keep_hbm, cnt_hbm, zero_hbm, out0, out1, out2,
                 keep_v, cnt_v, r0, r1, r2):
    @pl.when((lax.axis_index("c") == 0) & (lax.axis_index("s") == 0))
    def _():
        pltpu.sync_copy(keep_hbm, keep_v)
        pltpu.sync_copy(cnt_hbm, cnt_v)
        pltpu.sync_copy(zero_hbm, r0)
        pltpu.sync_copy(zero_hbm, r1)
        pltpu.sync_copy(zero_hbm, r2)
        cnt = cnt_v[...]
        cum_start = plsc.cumsum(cnt) - cnt
        lane = jax.lax.iota(jnp.int32, 16)
        zero16 = jnp.zeros((16,), jnp.int32)
        for c in range(_BC):
            cs = jnp.sum(jnp.where(lane == c, cum_start, 0))
            bval = jnp.full((16,), c >> 3, jnp.int32)
            cval = jnp.full((16,), c & 7, jnp.int32)
            for k in range(13):
                sel = keep_v[c, pl.ds(16 * k, 16)]
                r0[pl.ds(cs + 16 * k, 16)] = bval
                r1[pl.ds(cs + 16 * k, 16)] = cval
                r2[pl.ds(cs + 16 * k, 16)] = sel
        total = jnp.sum(cnt)
        for t in range(14):
            r0[pl.ds(total + 16 * t, 16)] = zero16
            r1[pl.ds(total + 16 * t, 16)] = zero16
            r2[pl.ds(total + 16 * t, 16)] = zero16
        pltpu.sync_copy(r0.at[pl.ds(0, _P)], out0)
        pltpu.sync_copy(r1.at[pl.ds(0, _P)], out1)
        pltpu.sync_copy(r2.at[pl.ds(0, _P)], out2)


def kernel(boxes, scores):
    B, C, N = scores.shape
    sc = scores.reshape(B * C, N)
    sc = jnp.pad(sc, ((0, 0), (0, _NPAD - N)))
    bx = jnp.pad(boxes, ((0, 0), (0, _NPAD - N), (0, 0)))
    x1 = jnp.repeat(bx[:, :, 1], C, axis=0)
    y1 = jnp.repeat(bx[:, :, 0], C, axis=0)
    x2 = jnp.repeat(bx[:, :, 3], C, axis=0)
    y2 = jnp.repeat(bx[:, :, 2], C, axis=0)

    tmat = jnp.stack([x1[0], y1[0], x2[0], y2[0],
                      x1[8], y1[8], x2[8], y2[8]], axis=1)

    keep, cnt = pl.pallas_call(
        _nms_kernel,
        out_shape=[
            jax.ShapeDtypeStruct((_BC, _KPAD), jnp.int32),
            jax.ShapeDtypeStruct((_BC, 1), jnp.int32),
        ],
    )(x1, y1, x2, y2, sc, tmat)

    zero = jnp.zeros((_PBUF,), jnp.int32)
    pack = pl.kernel(
        _pack_kernel,
        mesh=plsc.VectorSubcoreMesh(core_axis_name="c", subcore_axis_name="s"),
        compiler_params=pltpu.CompilerParams(needs_layout_passes=False),
        out_type=[jax.ShapeDtypeStruct((_P,), jnp.int32)] * 3,
        scratch_types=[
            pltpu.VMEM((_BC, _KPAD), jnp.int32),
            pltpu.VMEM((_BC,), jnp.int32),
            pltpu.VMEM((_PBUF,), jnp.int32),
            pltpu.VMEM((_PBUF,), jnp.int32),
            pltpu.VMEM((_PBUF,), jnp.int32),
        ],
    )
    o0, o1, o2 = pack(keep, cnt.reshape(_BC), zero)
    out = jnp.stack([o0, o1, o2], axis=1)

    out_dtype = jnp.asarray(np.zeros((1,), dtype=np.int64)).dtype
    return out.astype(out_dtype)


# speedup vs baseline: 1.8574x; 1.8574x over previous
import jax
import jax.numpy as jnp
import numpy as np
from jax import lax
from jax.experimental import pallas as pl
from jax.experimental.pallas import tpu as pltpu
from jax.experimental.pallas import tpu_sc as plsc

_IOU_T = 0.5
_SCORE_T = 0.5
_MAX_OUT = 200
_N = 5000
_NPAD = 5120
_BC = 16
_KPAD = 256
_P = _BC * _MAX_OUT
_PBUF = _P + 256


def _nms_kernel(x1_ref, y1_ref, x2_ref, y2_ref, sc_ref, keep_ref, cnt_ref):
    x1 = x1_ref[...]
    y1 = y1_ref[...]
    x2 = x2_ref[...]
    y2 = y2_ref[...]
    sc = sc_ref[...]
    areas = (x2 - x1) * (y2 - y1)

    iota_n = jax.lax.broadcasted_iota(jnp.int32, (_BC, _NPAD), 1)
    col_iota = jax.lax.broadcasted_iota(jnp.int32, (_BC, _KPAD), 1)

    scm0 = jnp.where(sc > _SCORE_T, sc, -1.0)
    keep0 = jnp.zeros((_BC, _KPAD), jnp.int32)
    cnt0 = jnp.zeros((_BC, 1), jnp.int32)

    def body(k, state):
        scm, keep, cnt = state
        m = jnp.max(scm, axis=1, keepdims=True)
        found = m > _SCORE_T
        idx = jnp.min(jnp.where(scm == m, iota_n, _NPAD), axis=1, keepdims=True)
        onehot = iota_n == idx
        bx1 = jnp.sum(jnp.where(onehot, x1, 0.0), axis=1, keepdims=True)
        by1 = jnp.sum(jnp.where(onehot, y1, 0.0), axis=1, keepdims=True)
        bx2 = jnp.sum(jnp.where(onehot, x2, 0.0), axis=1, keepdims=True)
        by2 = jnp.sum(jnp.where(onehot, y2, 0.0), axis=1, keepdims=True)
        barea = (bx2 - bx1) * (by2 - by1)
        w = jnp.maximum(jnp.minimum(x2, bx2) - jnp.maximum(x1, bx1), 0.0)
        h = jnp.maximum(jnp.minimum(y2, by2) - jnp.maximum(y1, by1), 0.0)
        inter = w * h
        union = areas + barea - inter
        sup = inter / union > _IOU_T
        scm = jnp.where((sup | onehot) & found, -1.0, scm)
        keep = keep + jnp.where((col_iota == k) & found, idx, 0)
        cnt = cnt + found.astype(jnp.int32)
        return scm, keep, cnt

    _, keep, cnt = jax.lax.fori_loop(0, _MAX_OUT, body, (scm0, keep0, cnt0),
                                     unroll=4)
    keep_ref[...] = keep
    cnt_ref[...] = cnt


def _pack_kernel(keep_hbm, cnt_hbm, zero_hbm, out0, out1, out2,
                 keep_v, cnt_v, r0, r1, r2):
    @pl.when((lax.axis_index("c") == 0) & (lax.axis_index("s") == 0))
    def _():
        pltpu.sync_copy(keep_hbm, keep_v)
        pltpu.sync_copy(cnt_hbm, cnt_v)
        pltpu.sync_copy(zero_hbm, r0)
        pltpu.sync_copy(zero_hbm, r1)
        pltpu.sync_copy(zero_hbm, r2)
        cnt = cnt_v[...]
        cum_start = plsc.cumsum(cnt) - cnt
        lane = jax.lax.iota(jnp.int32, 16)
        zero16 = jnp.zeros((16,), jnp.int32)
        for c in range(_BC):
            cs = jnp.sum(jnp.where(lane == c, cum_start, 0))
            bval = jnp.full((16,), c >> 3, jnp.int32)
            cval = jnp.full((16,), c & 7, jnp.int32)
            for k in range(13):
                sel = keep_v[c, pl.ds(16 * k, 16)]
                r0[pl.ds(cs + 16 * k, 16)] = bval
                r1[pl.ds(cs + 16 * k, 16)] = cval
                r2[pl.ds(cs + 16 * k, 16)] = sel
        total = jnp.sum(cnt)
        for t in range(14):
            r0[pl.ds(total + 16 * t, 16)] = zero16
            r1[pl.ds(total + 16 * t, 16)] = zero16
            r2[pl.ds(total + 16 * t, 16)] = zero16
        pltpu.sync_copy(r0.at[pl.ds(0, _P)], out0)
        pltpu.sync_copy(r1.at[pl.ds(0, _P)], out1)
        pltpu.sync_copy(r2.at[pl.ds(0, _P)], out2)


def kernel(boxes, scores):
    B, C, N = scores.shape
    sc = scores.reshape(B * C, N)
    sc = jnp.pad(sc, ((0, 0), (0, _NPAD - N)))
    bx = jnp.pad(boxes, ((0, 0), (0, _NPAD - N), (0, 0)))
    x1 = jnp.repeat(bx[:, :, 1], C, axis=0)
    y1 = jnp.repeat(bx[:, :, 0], C, axis=0)
    x2 = jnp.repeat(bx[:, :, 3], C, axis=0)
    y2 = jnp.repeat(bx[:, :, 2], C, axis=0)

    keep, cnt = pl.pallas_call(
        _nms_kernel,
        out_shape=[
            jax.ShapeDtypeStruct((_BC, _KPAD), jnp.int32),
            jax.ShapeDtypeStruct((_BC, 1), jnp.int32),
        ],
    )(x1, y1, x2, y2, sc)

    zero = jnp.zeros((_PBUF,), jnp.int32)
    pack = pl.kernel(
        _pack_kernel,
        mesh=plsc.VectorSubcoreMesh(core_axis_name="c", subcore_axis_name="s"),
        compiler_params=pltpu.CompilerParams(needs_layout_passes=False),
        out_type=[jax.ShapeDtypeStruct((_P,), jnp.int32)] * 3,
        scratch_types=[
            pltpu.VMEM((_BC, _KPAD), jnp.int32),
            pltpu.VMEM((_BC,), jnp.int32),
            pltpu.VMEM((_PBUF,), jnp.int32),
            pltpu.VMEM((_PBUF,), jnp.int32),
            pltpu.VMEM((_PBUF,), jnp.int32),
        ],
    )
    o0, o1, o2 = pack(keep, cnt.reshape(_BC), zero)
    out = jnp.stack([o0, o1, o2], axis=1)

    out_dtype = jnp.asarray(np.zeros((1,), dtype=np.int64)).dtype
    return out.astype(out_dtype)
